# baseline (speedup 1.0000x reference)
import jax
import jax.numpy as jnp
from jax.experimental import pallas as pl

X_DIM = 512
Y_DIM = 512
EMBED_DIM = 128
DBLK = 8


def _body(x_ref, y_ref, o_ref):
    o_ref[0] = y_ref[...][:, :, None] + x_ref[...][:, None, :]


def kernel(x_table, y_table):
    xT = x_table.T
    yT = y_table.T
    out = pl.pallas_call(
        _body,
        grid=(EMBED_DIM // DBLK,),
        in_specs=[
            pl.BlockSpec((DBLK, X_DIM), lambda i: (i, 0)),
            pl.BlockSpec((DBLK, Y_DIM), lambda i: (i, 0)),
        ],
        out_specs=pl.BlockSpec((1, DBLK, Y_DIM, X_DIM), lambda i: (0, i, 0, 0)),
        out_shape=jax.ShapeDtypeStruct((1, EMBED_DIM, Y_DIM, X_DIM), jnp.float32),
    )(xT, yT)
    return out
